# baseline (speedup 1.0000x reference)
import functools

import jax
import jax.numpy as jnp
from jax import lax
from jax.experimental import pallas as pl
from jax.experimental.pallas import tpu as pltpu
from jax.experimental.pallas import tpu_sc as plsc

_BLK = 128
_DPAD = 16384


def _sc_info():
    try:
        info = plsc.get_sparse_core_info()
        return info.num_cores, info.num_subcores
    except Exception:
        return 2, 16


@functools.lru_cache(maxsize=None)
def _sc_deg_kernel(n, bpw):
    nc, ns = _sc_info()
    dpt = _DPAD // ns

    mesh = plsc.VectorSubcoreMesh(core_axis_name="c", subcore_axis_name="s")
    out_t = jax.ShapeDtypeStruct((nc * _DPAD,), jnp.float32)
    scratch = [
        pltpu.VMEM((_BLK,), jnp.int32),
        pltpu.VMEM((_BLK,), jnp.float32),
        pltpu.VMEM((dpt,), jnp.float32),
        pltpu.VMEM_SHARED((_DPAD,), jnp.float32),
    ]

    def body(dstr, out_deg, dst_v, ones_v, stage, deg_sh):
        c = lax.axis_index("c")
        s = lax.axis_index("s")
        wid = s * nc + c

        zero16 = jnp.zeros((16,), jnp.float32)
        one16 = jnp.ones((16,), jnp.float32)

        def fill_body(i, carry):
            stage[pl.ds(i * 16, 16)] = zero16
            return carry

        lax.fori_loop(0, dpt // 16, fill_body, 0)
        for i in range(_BLK // 16):
            ones_v[pl.ds(i * 16, 16)] = one16

        pltpu.sync_copy(stage, deg_sh.at[pl.ds(s * dpt, dpt)])
        plsc.subcore_barrier()

        e0 = wid * bpw * _BLK

        def ebody(g, carry):
            pltpu.sync_copy(dstr.at[pl.ds(e0 + g * _BLK, _BLK)], dst_v)
            pltpu.sync_copy(ones_v, deg_sh.at[dst_v], add=True)
            return carry

        lax.fori_loop(0, bpw, ebody, 0)
        plsc.subcore_barrier()

        pltpu.sync_copy(deg_sh.at[pl.ds(s * dpt, dpt)], stage)
        pltpu.sync_copy(stage, out_deg.at[pl.ds(c * _DPAD + s * dpt, dpt)])

    return pl.kernel(body, out_type=out_t, mesh=mesh, scratch_types=scratch)


@functools.lru_cache(maxsize=None)
def _sc_edge_kernel(n, d, bpw):
    nc, ns = _sc_info()
    npad = ((n + 1 + ns * _BLK - 1) // (ns * _BLK)) * (ns * _BLK)
    zpt = npad // ns
    assert zpt % _BLK == 0

    assert bpw % 4 == 0

    mesh = plsc.VectorSubcoreMesh(core_axis_name="c", subcore_axis_name="s")
    out_t = jax.ShapeDtypeStruct((nc, npad, d), jnp.float32)
    scratch = [pltpu.VMEM((_BLK,), jnp.int32) for _ in range(4)]
    scratch += [pltpu.VMEM((_BLK,), jnp.int32) for _ in range(4)]
    scratch += [pltpu.VMEM((_BLK, d), jnp.float32) for _ in range(2)]
    scratch += [
        pltpu.VMEM((zpt,), jnp.float32),
        pltpu.VMEM((zpt,), jnp.float32),
        pltpu.VMEM_SHARED((npad, d), jnp.float32),
    ]
    scratch += [pltpu.SemaphoreType.DMA for _ in range(8)]

    def body(tbl, srcr, dstr, degp, out_agg, *rest):
        (sb0, sb1, sb2, sb3, db0, db1, db2, db3, r0b, r1b,
         dg0, dinv, agg_sh, i0, i1, i2, i3, g0s, g1s, s0s, s1s) = rest
        src_b = [sb0, sb1, sb2, sb3]
        dst_b = [db0, db1, db2, db3]
        rows_b = [r0b, r1b]
        isem = [i0, i1, i2, i3]
        gsem = [g0s, g1s]
        ssem = [s0s, s1s]
        c = lax.axis_index("c")
        s = lax.axis_index("s")
        wid = s * nc + c
        rows = rows_b[0]

        zero16 = jnp.zeros((16,), jnp.float32)

        def zrow_body(r, carry):
            for q in range(d // 16):
                rows[r, pl.ds(q * 16, 16)] = zero16
            return carry

        lax.fori_loop(0, _BLK, zrow_body, 0)

        zb = s * zpt

        def zbody(k, carry):
            pltpu.sync_copy(rows, agg_sh.at[pl.ds(zb + k * _BLK, _BLK)])
            return carry

        lax.fori_loop(0, zpt // _BLK, zbody, 0)

        pltpu.sync_copy(degp.at[pl.ds(s * zpt, zpt)], dg0)
        pltpu.sync_copy(degp.at[pl.ds(_DPAD + s * zpt, zpt)], dinv)

        def dbody(i, carry):
            dtot = dg0[pl.ds(i * 16, 16)] + dinv[pl.ds(i * 16, 16)]
            dinv[pl.ds(i * 16, 16)] = 1.0 / jnp.maximum(dtot, 1.0)
            return carry

        lax.fori_loop(0, zpt // 16, dbody, 0)

        plsc.subcore_barrier()

        e0 = wid * bpw * _BLK

        def issue_idx(g, b):
            off = e0 + g * _BLK
            pltpu.async_copy(srcr.at[pl.ds(off, _BLK)], src_b[b], isem[b])
            pltpu.async_copy(dstr.at[pl.ds(off, _BLK)], dst_b[b], isem[b])

        def wait_idx(b):
            pltpu.make_async_copy(srcr.at[pl.ds(0, _BLK)], src_b[b],
                                  isem[b]).wait()
            pltpu.make_async_copy(dstr.at[pl.ds(0, _BLK)], dst_b[b],
                                  isem[b]).wait()

        def drain_scat(rb):
            pltpu.make_async_copy(rows_b[rb], agg_sh.at[pl.ds(0, _BLK)],
                                  ssem[rb]).wait()

        issue_idx(0, 0)
        issue_idx(1, 1)

        def ebody(g4, carry):
            for b in range(4):
                g = g4 * 4 + b
                rb = b % 2
                pb = (b + 2) % 4
                if b >= 2:
                    drain_scat(rb)
                    @pl.when(g4 < (bpw // 4) - 1)
                    def _():
                        issue_idx(g + 2, pb)
                else:
                    @pl.when(g4 > 0)
                    def _():
                        drain_scat(rb)
                    issue_idx(g + 2, pb)

                wait_idx(b)
                pltpu.async_copy(tbl.at[src_b[b]], rows_b[rb],
                                 gsem[rb]).wait()
                pltpu.async_copy(rows_b[rb], agg_sh.at[dst_b[b]], ssem[rb],
                                 add=True)
            return carry

        lax.fori_loop(0, bpw // 4, ebody, 0)
        drain_scat(0)
        drain_scat(1)
        plsc.subcore_barrier()

        def obody(k, carry):
            r0 = zb + k * _BLK
            pltpu.sync_copy(agg_sh.at[pl.ds(r0, _BLK)], rows)

            def nbody(r16, carry2):
                dv16 = dinv[pl.ds(k * _BLK + r16 * 16, 16)]
                for j in range(16):
                    dv = jnp.full((16,), dv16[j], jnp.float32)
                    row = r16 * 16 + j
                    for q in range(d // 16):
                        rows[row, pl.ds(q * 16, 16)] = (
                            rows[row, pl.ds(q * 16, 16)] * dv)
                return carry2

            lax.fori_loop(0, _BLK // 16, nbody, 0)
            pltpu.sync_copy(rows, out_agg.at[c, pl.ds(r0, _BLK)])
            return carry

        lax.fori_loop(0, zpt // _BLK, obody, 0)

    return pl.kernel(body, out_type=out_t, mesh=mesh, scratch_types=scratch)


def _tc_prep(x, ws, wn, b2d):
    n, d = x.shape

    def tc_prep_body(x_ref, ws_ref, wn_ref, b_ref, hs_ref, hn_ref):
        xv = x_ref[...]
        hs_ref[...] = (jnp.dot(xv, ws_ref[...],
                               preferred_element_type=jnp.float32) + b_ref[...])
        hn_ref[...] = jnp.dot(xv, wn_ref[...],
                              preferred_element_type=jnp.float32)

    return pl.pallas_call(
        tc_prep_body,
        out_shape=[jax.ShapeDtypeStruct((n, d), jnp.float32)] * 2,
    )(x, ws, wn, b2d)


def _tc_mid(aggp, hs_prev, ws, wn, b2d):
    n, d = hs_prev.shape

    def tc_mid_body(ap_r, hsp_r, ws_ref, wn_ref, b_ref, hs_ref, hn_ref):
        agg = ap_r[0, :n, :] + ap_r[1, :n, :]
        h = jnp.maximum(hsp_r[...] + agg, 0.0)
        hs_ref[...] = (jnp.dot(h, ws_ref[...],
                               preferred_element_type=jnp.float32) + b_ref[...])
        hn_ref[...] = jnp.dot(h, wn_ref[...],
                              preferred_element_type=jnp.float32)

    return pl.pallas_call(
        tc_mid_body,
        out_shape=[jax.ShapeDtypeStruct((n, d), jnp.float32)] * 2,
    )(aggp, hs_prev, ws, wn, b2d)


def _tc_final(aggp, hs_prev):
    n, d = hs_prev.shape

    def tc_final_body(ap_r, hsp_r, out_ref):
        agg = ap_r[0, :n, :] + ap_r[1, :n, :]
        h = jnp.maximum(hsp_r[...] + agg, 0.0)
        out_ref[...] = jnp.sum(h, axis=0, keepdims=True) * (1.0 / n)

    return pl.pallas_call(
        tc_final_body,
        out_shape=jax.ShapeDtypeStruct((1, d), jnp.float32),
    )(aggp, hs_prev)


def kernel(x, edge_index, W1_self, W1_nbr, b1, W2_self, W2_nbr, b2):
    n, d = x.shape
    e = edge_index.shape[1]
    nc, ns = _sc_info()
    nw = nc * ns
    bpw = -(-e // (nw * _BLK))
    bpw = ((bpw + 7) // 8) * 8
    e_pad = bpw * nw * _BLK
    pad = e_pad - e

    src = edge_index[0]
    dst = edge_index[1]
    if pad:
        dummy = n + (jnp.arange(pad, dtype=dst.dtype) % 128)
        src = jnp.concatenate([src, jnp.zeros((pad,), src.dtype)])
        dst = jnp.concatenate([dst, dummy])

    b1r = b1.reshape(1, d)
    b2r = b2.reshape(1, d)

    degp = _sc_deg_kernel(n, bpw)(dst)
    hs1, hn1 = _tc_prep(x, W1_self, W1_nbr, b1r)
    agg1 = _sc_edge_kernel(n, d, bpw)(hn1, src, dst, degp)
    hs2, hn2 = _tc_mid(agg1, hs1, W2_self, W2_nbr, b2r)
    agg2 = _sc_edge_kernel(n, d, bpw)(hn2, src, dst, degp)
    return _tc_final(agg2, hs2)


# speedup vs baseline: 1.1221x; 1.1221x over previous
import functools

import jax
import jax.numpy as jnp
from jax import lax
from jax.experimental import pallas as pl
from jax.experimental.pallas import tpu as pltpu
from jax.experimental.pallas import tpu_sc as plsc

_BLK = 128
_DPAD = 16384


def _sc_info():
    try:
        info = plsc.get_sparse_core_info()
        return info.num_cores, info.num_subcores
    except Exception:
        return 2, 16


@functools.lru_cache(maxsize=None)
def _sc_deg_kernel(n, bpw):
    nc, ns = _sc_info()
    dpt = _DPAD // ns

    mesh = plsc.VectorSubcoreMesh(core_axis_name="c", subcore_axis_name="s")
    out_t = jax.ShapeDtypeStruct((nc * _DPAD,), jnp.float32)
    scratch = [
        pltpu.VMEM((_BLK,), jnp.int32),
        pltpu.VMEM((_BLK,), jnp.float32),
        pltpu.VMEM((dpt,), jnp.float32),
        pltpu.VMEM_SHARED((_DPAD,), jnp.float32),
    ]

    def body(dstr, out_deg, dst_v, ones_v, stage, deg_sh):
        c = lax.axis_index("c")
        s = lax.axis_index("s")
        wid = s * nc + c

        zero16 = jnp.zeros((16,), jnp.float32)
        one16 = jnp.ones((16,), jnp.float32)

        def fill_body(i, carry):
            stage[pl.ds(i * 16, 16)] = zero16
            return carry

        lax.fori_loop(0, dpt // 16, fill_body, 0)
        for i in range(_BLK // 16):
            ones_v[pl.ds(i * 16, 16)] = one16

        pltpu.sync_copy(stage, deg_sh.at[pl.ds(s * dpt, dpt)])
        plsc.subcore_barrier()

        e0 = wid * bpw * _BLK

        def ebody(g, carry):
            pltpu.sync_copy(dstr.at[pl.ds(e0 + g * _BLK, _BLK)], dst_v)
            pltpu.sync_copy(ones_v, deg_sh.at[dst_v], add=True)
            return carry

        lax.fori_loop(0, bpw, ebody, 0)
        plsc.subcore_barrier()

        pltpu.sync_copy(deg_sh.at[pl.ds(s * dpt, dpt)], stage)
        pltpu.sync_copy(stage, out_deg.at[pl.ds(c * _DPAD + s * dpt, dpt)])

    return pl.kernel(body, out_type=out_t, mesh=mesh, scratch_types=scratch)


@functools.lru_cache(maxsize=None)
def _sc_edge_kernel(n, d, b0, b1):
    nc, ns = _sc_info()
    npad = ((n + 1 + ns * _BLK - 1) // (ns * _BLK)) * (ns * _BLK)
    zpt = npad // ns
    assert zpt % _BLK == 0

    assert b0 % 4 == 0 and b1 % 4 == 0

    mesh = plsc.VectorSubcoreMesh(core_axis_name="c", subcore_axis_name="s")
    out_t = jax.ShapeDtypeStruct((nc, npad, d), jnp.float32)
    scratch = [pltpu.VMEM((_BLK,), jnp.int32) for _ in range(4)]
    scratch += [pltpu.VMEM((_BLK,), jnp.int32) for _ in range(4)]
    scratch += [pltpu.VMEM((_BLK, d), jnp.float32) for _ in range(2)]
    scratch += [
        pltpu.VMEM((zpt,), jnp.float32),
        pltpu.VMEM((zpt,), jnp.float32),
        pltpu.VMEM_SHARED((npad, d), jnp.float32),
    ]
    scratch += [pltpu.SemaphoreType.DMA for _ in range(8)]

    def body(tbl, srcr, dstr, degp, out_agg, *rest):
        (sb0, sb1, sb2, sb3, db0, db1, db2, db3, r0b, r1b,
         dg0, dinv, agg_sh, i0, i1, i2, i3, g0s, g1s, s0s, s1s) = rest
        src_b = [sb0, sb1, sb2, sb3]
        dst_b = [db0, db1, db2, db3]
        rows_b = [r0b, r1b]
        isem = [i0, i1, i2, i3]
        gsem = [g0s, g1s]
        ssem = [s0s, s1s]
        c = lax.axis_index("c")
        s = lax.axis_index("s")
        wid = s * nc + c
        rows = rows_b[0]

        zero16 = jnp.zeros((16,), jnp.float32)

        def zrow_body(r, carry):
            for q in range(d // 16):
                rows[r, pl.ds(q * 16, 16)] = zero16
            return carry

        lax.fori_loop(0, _BLK, zrow_body, 0)

        zb = s * zpt

        def zbody(k, carry):
            pltpu.sync_copy(rows, agg_sh.at[pl.ds(zb + k * _BLK, _BLK)])
            return carry

        lax.fori_loop(0, zpt // _BLK, zbody, 0)

        pltpu.sync_copy(degp.at[pl.ds(s * zpt, zpt)], dg0)
        pltpu.sync_copy(degp.at[pl.ds(_DPAD + s * zpt, zpt)], dinv)

        def dbody(i, carry):
            dtot = dg0[pl.ds(i * 16, 16)] + dinv[pl.ds(i * 16, 16)]
            dinv[pl.ds(i * 16, 16)] = 1.0 / jnp.maximum(dtot, 1.0)
            return carry

        lax.fori_loop(0, zpt // 16, dbody, 0)

        plsc.subcore_barrier()

        bc = jnp.where(c == 0, b0, b1)
        g4max = bc // 4
        e0 = (c * ns * b0 + s * bc) * _BLK

        def issue_idx(g, b):
            off = e0 + g * _BLK
            pltpu.async_copy(srcr.at[pl.ds(off, _BLK)], src_b[b], isem[b])
            pltpu.async_copy(dstr.at[pl.ds(off, _BLK)], dst_b[b], isem[b])

        def wait_idx(b):
            pltpu.make_async_copy(srcr.at[pl.ds(0, _BLK)], src_b[b],
                                  isem[b]).wait()
            pltpu.make_async_copy(dstr.at[pl.ds(0, _BLK)], dst_b[b],
                                  isem[b]).wait()

        def drain_scat(rb):
            pltpu.make_async_copy(rows_b[rb], agg_sh.at[pl.ds(0, _BLK)],
                                  ssem[rb]).wait()

        issue_idx(0, 0)
        issue_idx(1, 1)

        def ebody(g4, carry):
            for b in range(4):
                g = g4 * 4 + b
                rb = b % 2
                pb = (b + 2) % 4
                if b >= 2:
                    drain_scat(rb)
                    @pl.when(g4 < g4max - 1)
                    def _():
                        issue_idx(g + 2, pb)
                else:
                    @pl.when(g4 > 0)
                    def _():
                        drain_scat(rb)
                    issue_idx(g + 2, pb)

                wait_idx(b)
                pltpu.async_copy(tbl.at[src_b[b]], rows_b[rb],
                                 gsem[rb]).wait()
                pltpu.async_copy(rows_b[rb], agg_sh.at[dst_b[b]], ssem[rb],
                                 add=True)
            return carry

        lax.fori_loop(0, g4max, ebody, 0)
        drain_scat(0)
        drain_scat(1)
        plsc.subcore_barrier()

        def obody(k, carry):
            r0 = zb + k * _BLK
            pltpu.sync_copy(agg_sh.at[pl.ds(r0, _BLK)], rows)

            def nbody(r16, carry2):
                dv16 = dinv[pl.ds(k * _BLK + r16 * 16, 16)]
                for j in range(16):
                    dv = jnp.full((16,), dv16[j], jnp.float32)
                    row = r16 * 16 + j
                    for q in range(d // 16):
                        rows[row, pl.ds(q * 16, 16)] = (
                            rows[row, pl.ds(q * 16, 16)] * dv)
                return carry2

            lax.fori_loop(0, _BLK // 16, nbody, 0)
            pltpu.sync_copy(rows, out_agg.at[c, pl.ds(r0, _BLK)])
            return carry

        lax.fori_loop(0, zpt // _BLK, obody, 0)

    return pl.kernel(body, out_type=out_t, mesh=mesh, scratch_types=scratch)


def _tc_prep(x, ws, wn, b2d):
    n, d = x.shape

    def tc_prep_body(x_ref, ws_ref, wn_ref, b_ref, hs_ref, hn_ref):
        xv = x_ref[...]
        hs_ref[...] = (jnp.dot(xv, ws_ref[...],
                               preferred_element_type=jnp.float32) + b_ref[...])
        hn_ref[...] = jnp.dot(xv, wn_ref[...],
                              preferred_element_type=jnp.float32)

    return pl.pallas_call(
        tc_prep_body,
        out_shape=[jax.ShapeDtypeStruct((n, d), jnp.float32)] * 2,
    )(x, ws, wn, b2d)


def _tc_mid(aggp, hs_prev, ws, wn, b2d):
    n, d = hs_prev.shape

    def tc_mid_body(ap_r, hsp_r, ws_ref, wn_ref, b_ref, hs_ref, hn_ref):
        agg = ap_r[0, :n, :] + ap_r[1, :n, :]
        h = jnp.maximum(hsp_r[...] + agg, 0.0)
        hs_ref[...] = (jnp.dot(h, ws_ref[...],
                               preferred_element_type=jnp.float32) + b_ref[...])
        hn_ref[...] = jnp.dot(h, wn_ref[...],
                              preferred_element_type=jnp.float32)

    return pl.pallas_call(
        tc_mid_body,
        out_shape=[jax.ShapeDtypeStruct((n, d), jnp.float32)] * 2,
    )(aggp, hs_prev, ws, wn, b2d)


def _tc_final(aggp, hs_prev):
    n, d = hs_prev.shape

    def tc_final_body(ap_r, hsp_r, out_ref):
        agg = ap_r[0, :n, :] + ap_r[1, :n, :]
        h = jnp.maximum(hsp_r[...] + agg, 0.0)
        out_ref[...] = jnp.sum(h, axis=0, keepdims=True) * (1.0 / n)

    return pl.pallas_call(
        tc_final_body,
        out_shape=jax.ShapeDtypeStruct((1, d), jnp.float32),
    )(aggp, hs_prev)


def kernel(x, edge_index, W1_self, W1_nbr, b1, W2_self, W2_nbr, b2):
    n, d = x.shape
    e = edge_index.shape[1]
    nc, ns = _sc_info()
    nw = nc * ns
    bpw = -(-e // (nw * _BLK))
    bpw = ((bpw + 7) // 8) * 8
    e_pad = bpw * nw * _BLK
    pad = e_pad - e

    src = edge_index[0]
    dst = edge_index[1]
    if pad:
        dummy = n + (jnp.arange(pad, dtype=dst.dtype) % 128)
        src = jnp.concatenate([src, jnp.zeros((pad,), src.dtype)])
        dst = jnp.concatenate([dst, dummy])

    b1r = b1.reshape(1, d)
    b2r = b2.reshape(1, d)

    total_b = 2 * bpw
    b1_blocks = max(4, ((total_b * 36 // 160) // 4) * 4)
    b0_blocks = total_b - b1_blocks

    degp = _sc_deg_kernel(n, bpw)(dst)
    hs1, hn1 = _tc_prep(x, W1_self, W1_nbr, b1r)
    agg1 = _sc_edge_kernel(n, d, b0_blocks, b1_blocks)(hn1, src, dst, degp)
    hs2, hn2 = _tc_mid(agg1, hs1, W2_self, W2_nbr, b2r)
    agg2 = _sc_edge_kernel(n, d, b0_blocks, b1_blocks)(hn2, src, dst, degp)
    return _tc_final(agg2, hs2)
